# baseline (device time: 212411 ns/iter reference)
import jax
import jax.numpy as jnp
from jax import lax
from jax.experimental import pallas as pl
from jax.experimental.pallas import tpu as pltpu

N_DEV = 16
S = 4

N_HOPS = {}
for p in range(S):
    N_HOPS[("R", p)] = 8 if p < S // 2 else 7
    N_HOPS[("L", p)] = 7 if p < S // 2 else 8
KEYS = tuple(N_HOPS)
ROUND_ORDER = tuple((d, p) for p in range(S) for d in ("R", "L"))


def kernel(x):
    m_per, n = x.shape
    m_piece = m_per // S

    def body(x_ref, out_ref, *sems):
        sem_pairs = {k: (sems[2 * i], sems[2 * i + 1]) for i, k in enumerate(KEYS)}
        my = lax.axis_index("i")
        left = (my - 1 + N_DEV) % N_DEV
        right = (my + 1) % N_DEV

        barrier_sem = pltpu.get_barrier_semaphore()
        for nbr in (left, right):
            pl.semaphore_signal(
                barrier_sem, inc=1,
                device_id=(nbr,), device_id_type=pl.DeviceIdType.MESH,
            )
        pl.semaphore_wait(barrier_sem, 2)

        out_ref[pl.ds(my * m_per, m_per), :] = x_ref[:, :].astype(jnp.bfloat16)

        def make_rdma(key, hop):
            dirn, piece = key
            origin = (my - hop + N_DEV) % N_DEV if dirn == "R" else (my + hop) % N_DEV
            rows = origin * m_per + piece * m_piece
            send_sems, recv_sems = sem_pairs[key]
            return pltpu.make_async_remote_copy(
                src_ref=out_ref.at[pl.ds(rows, m_piece)],
                dst_ref=out_ref.at[pl.ds(rows, m_piece)],
                send_sem=send_sems.at[hop],
                recv_sem=recv_sems.at[hop],
                device_id=(right if dirn == "R" else left,),
                device_id_type=pl.DeviceIdType.MESH,
            )

        descs = {}
        for key in ROUND_ORDER:
            d = make_rdma(key, 0)
            d.start()
            descs[key + (0,)] = d

        for h in range(1, max(N_HOPS.values())):
            for key in ROUND_ORDER:
                if h < N_HOPS[key]:
                    descs[key + (h - 1,)].wait_recv()
                    d = make_rdma(key, h)
                    d.start()
                    descs[key + (h,)] = d

        for key in ROUND_ORDER:
            descs[key + (N_HOPS[key] - 1,)].wait_recv()
        for key in KEYS:
            for h in range(N_HOPS[key]):
                descs[key + (h,)].wait_send()

    out_shape = jax.ShapeDtypeStruct((N_DEV * m_per, n), jnp.bfloat16)
    scratch = []
    for key in KEYS:
        scratch.append(pltpu.SemaphoreType.DMA((N_HOPS[key],)))
        scratch.append(pltpu.SemaphoreType.DMA((N_HOPS[key],)))
    return pl.pallas_call(
        body,
        out_shape=out_shape,
        in_specs=[pl.BlockSpec(memory_space=pltpu.VMEM)],
        out_specs=pl.BlockSpec(memory_space=pltpu.VMEM),
        scratch_shapes=scratch,
        compiler_params=pltpu.CompilerParams(collective_id=0),
    )(x)


# device time: 201409 ns/iter; 1.0546x vs baseline; 1.0546x over previous
import jax
import jax.numpy as jnp
from jax import lax
from jax.experimental import pallas as pl
from jax.experimental.pallas import tpu as pltpu

N_DEV = 16
S = 4

N_HOPS = {}
for p in range(S):
    N_HOPS[("R", p)] = 8 if p < S // 2 else 7
    N_HOPS[("L", p)] = 7 if p < S // 2 else 8
KEYS = tuple(N_HOPS)
ROUND_ORDER = tuple((d, p) for p in range(S) for d in ("R", "L"))


def kernel(x):
    m_per, n = x.shape
    m_piece = m_per // S

    def body(x_ref, out_ref, own_ref, copy_sem, *sems):
        sem_pairs = {k: (sems[2 * i], sems[2 * i + 1]) for i, k in enumerate(KEYS)}
        my = lax.axis_index("i")
        left = (my - 1 + N_DEV) % N_DEV
        right = (my + 1) % N_DEV

        barrier_sem = pltpu.get_barrier_semaphore()
        for nbr in (left, right):
            pl.semaphore_signal(
                barrier_sem, inc=1,
                device_id=(nbr,), device_id_type=pl.DeviceIdType.MESH,
            )
        own_ref[:, :] = x_ref[:, :].astype(jnp.bfloat16)
        own_copy = pltpu.make_async_copy(
            own_ref, out_ref.at[pl.ds(my * m_per, m_per)], copy_sem
        )
        own_copy.start()
        pl.semaphore_wait(barrier_sem, 2)

        def make_rdma(key, hop):
            dirn, piece = key
            origin = (my - hop + N_DEV) % N_DEV if dirn == "R" else (my + hop) % N_DEV
            rows = origin * m_per + piece * m_piece
            send_sems, recv_sems = sem_pairs[key]
            src = own_ref.at[pl.ds(piece * m_piece, m_piece)] if hop == 0 \
                else out_ref.at[pl.ds(rows, m_piece)]
            return pltpu.make_async_remote_copy(
                src_ref=src,
                dst_ref=out_ref.at[pl.ds(rows, m_piece)],
                send_sem=send_sems.at[hop],
                recv_sem=recv_sems.at[hop],
                device_id=(right if dirn == "R" else left,),
                device_id_type=pl.DeviceIdType.MESH,
            )

        descs = {}
        for key in ROUND_ORDER:
            d = make_rdma(key, 0)
            d.start()
            descs[key + (0,)] = d

        for h in range(1, max(N_HOPS.values())):
            for key in ROUND_ORDER:
                if h < N_HOPS[key]:
                    descs[key + (h - 1,)].wait_recv()
                    d = make_rdma(key, h)
                    d.start()
                    descs[key + (h,)] = d

        for key in ROUND_ORDER:
            descs[key + (N_HOPS[key] - 1,)].wait_recv()
        own_copy.wait()
        for key in KEYS:
            for h in range(N_HOPS[key]):
                descs[key + (h,)].wait_send()

    out_shape = jax.ShapeDtypeStruct((N_DEV * m_per, n), jnp.bfloat16)
    scratch = [
        pltpu.VMEM((m_per, n), jnp.bfloat16),
        pltpu.SemaphoreType.DMA,
    ]
    for key in KEYS:
        scratch.append(pltpu.SemaphoreType.DMA((N_HOPS[key],)))
        scratch.append(pltpu.SemaphoreType.DMA((N_HOPS[key],)))
    return pl.pallas_call(
        body,
        out_shape=out_shape,
        in_specs=[pl.BlockSpec(memory_space=pltpu.VMEM)],
        out_specs=pl.BlockSpec(memory_space=pl.ANY),
        scratch_shapes=scratch,
        compiler_params=pltpu.CompilerParams(collective_id=0),
    )(x)
